# baseline (speedup 1.0000x reference)
import jax
import jax.numpy as jnp
from jax.experimental import pallas as pl
from jax.experimental.pallas import tpu as pltpu

_ROWS_PER_BLOCK = 128


def _exchange_body(m1_ref, m2_ref, x0_ref, x1_ref, y1_ref, y2_ref):
    m1 = m1_ref[...] > 0
    m2 = m2_ref[...] > 0
    x0 = x0_ref[...]
    x1 = x1_ref[...]
    y1_ref[...] = jnp.where(m1, x0, x1)
    y2_ref[...] = jnp.where(m2, x1, x0)


def kernel(x0, x1, bn1_weight, bn2_weight, bn_threshold):
    B, C, H, W = x0.shape
    rows = B * C
    cols = H * W
    x0f = x0.reshape(rows, cols)
    x1f = x1.reshape(rows, cols)

    m1c = (jnp.abs(bn1_weight) >= bn_threshold).astype(jnp.float32)
    m2c = (jnp.abs(bn2_weight) >= bn_threshold).astype(jnp.float32)
    m1r = jnp.tile(m1c, B).reshape(rows, 1)
    m2r = jnp.tile(m2c, B).reshape(rows, 1)

    R = _ROWS_PER_BLOCK
    grid = (rows // R,)
    y1, y2 = pl.pallas_call(
        _exchange_body,
        grid=grid,
        in_specs=[
            pl.BlockSpec((R, 1), lambda i: (i, 0)),
            pl.BlockSpec((R, 1), lambda i: (i, 0)),
            pl.BlockSpec((R, cols), lambda i: (i, 0)),
            pl.BlockSpec((R, cols), lambda i: (i, 0)),
        ],
        out_specs=[
            pl.BlockSpec((R, cols), lambda i: (i, 0)),
            pl.BlockSpec((R, cols), lambda i: (i, 0)),
        ],
        out_shape=[
            jax.ShapeDtypeStruct((rows, cols), x0.dtype),
            jax.ShapeDtypeStruct((rows, cols), x0.dtype),
        ],
        compiler_params=pltpu.CompilerParams(
            dimension_semantics=("arbitrary",),
        ),
    )(m1r, m2r, x0f, x1f)
    return (y1.reshape(B, C, H, W), y2.reshape(B, C, H, W))


# speedup vs baseline: 1.1338x; 1.1338x over previous
import jax
import jax.numpy as jnp
from jax.experimental import pallas as pl
from jax.experimental.pallas import tpu as pltpu

_CB = 96


def _exchange_body(m1_ref, m2_ref, x0_ref, x1_ref, y1_ref, y2_ref):
    m1 = m1_ref[...] > 0
    m2 = m2_ref[...] > 0
    x0 = x0_ref[...]
    x1 = x1_ref[...]
    y1_ref[...] = jnp.where(m1, x0, x1)
    y2_ref[...] = jnp.where(m2, x1, x0)


def kernel(x0, x1, bn1_weight, bn2_weight, bn_threshold):
    B, C, H, W = x0.shape
    m1 = (jnp.abs(bn1_weight) >= bn_threshold).astype(jnp.float32)
    m2 = (jnp.abs(bn2_weight) >= bn_threshold).astype(jnp.float32)
    m1 = m1.reshape(1, C, 1, 1)
    m2 = m2.reshape(1, C, 1, 1)

    CB = _CB
    grid = (B, C // CB)
    y1, y2 = pl.pallas_call(
        _exchange_body,
        grid=grid,
        in_specs=[
            pl.BlockSpec((1, CB, 1, 1), lambda b, j: (0, j, 0, 0)),
            pl.BlockSpec((1, CB, 1, 1), lambda b, j: (0, j, 0, 0)),
            pl.BlockSpec((1, CB, H, W), lambda b, j: (b, j, 0, 0)),
            pl.BlockSpec((1, CB, H, W), lambda b, j: (b, j, 0, 0)),
        ],
        out_specs=[
            pl.BlockSpec((1, CB, H, W), lambda b, j: (b, j, 0, 0)),
            pl.BlockSpec((1, CB, H, W), lambda b, j: (b, j, 0, 0)),
        ],
        out_shape=[
            jax.ShapeDtypeStruct((B, C, H, W), x0.dtype),
            jax.ShapeDtypeStruct((B, C, H, W), x0.dtype),
        ],
        compiler_params=pltpu.CompilerParams(
            dimension_semantics=("parallel", "parallel"),
        ),
    )(m1, m2, x0, x1)
    return (y1, y2)


# speedup vs baseline: 2.2481x; 1.9827x over previous
import jax
import jax.numpy as jnp
from jax import lax
from jax.experimental import pallas as pl
from jax.experimental.pallas import tpu as pltpu
from jax.experimental.pallas import tpu_sc as plsc

L = 16


def _sc_body(x0_hbm, x1_hbm, bn1_hbm, bn2_hbm, thr_hbm,
             y1_hbm, y2_hbm,
             w_v, thr_v, m1_v, m2_v, a_v, b_v, o1_v, o2_v):
    nc = 2
    wid = lax.axis_index("s") * nc + lax.axis_index("c")
    B, H, W, C = x0_hbm.shape
    planes_per_w = (B * H) // (nc * L)
    p0 = wid * planes_per_w

    pltpu.sync_copy(thr_hbm, thr_v)
    thr = thr_v[...]
    pltpu.sync_copy(bn1_hbm, w_v)
    for k in range(C // L):
        m1_v[pl.ds(k * L, L)] = jnp.where(
            jnp.abs(w_v[pl.ds(k * L, L)]) >= thr, 1.0, 0.0)
    pltpu.sync_copy(bn2_hbm, w_v)
    for k in range(C // L):
        m2_v[pl.ds(k * L, L)] = jnp.where(
            jnp.abs(w_v[pl.ds(k * L, L)]) >= thr, 1.0, 0.0)

    def plane(i):
        p = p0 + i
        b = p // H
        h = p % H
        pltpu.sync_copy(x0_hbm.at[b, h], a_v)
        pltpu.sync_copy(x1_hbm.at[b, h], b_v)

        def wrow(w):
            for k in range(C // L):
                sl = pl.ds(k * L, L)
                av = a_v[w, sl]
                bv = b_v[w, sl]
                m1 = m1_v[sl] > 0.5
                m2 = m2_v[sl] > 0.5
                o1_v[w, sl] = jnp.where(m1, av, bv)
                o2_v[w, sl] = jnp.where(m2, bv, av)

        pl.loop(0, W)(wrow)
        pltpu.sync_copy(o1_v, y1_hbm.at[b, h])
        pltpu.sync_copy(o2_v, y2_hbm.at[b, h])

    pl.loop(0, planes_per_w)(plane)


def kernel(x0, x1, bn1_weight, bn2_weight, bn_threshold):
    B, C, H, W = x0.shape
    x0t = jnp.transpose(x0, (0, 2, 3, 1))
    x1t = jnp.transpose(x1, (0, 2, 3, 1))
    thr = jnp.full((L,), bn_threshold, dtype=jnp.float32)

    mesh = plsc.VectorSubcoreMesh(core_axis_name="c", subcore_axis_name="s")
    run = pl.kernel(
        _sc_body,
        out_type=[
            jax.ShapeDtypeStruct((B, H, W, C), jnp.float32),
            jax.ShapeDtypeStruct((B, H, W, C), jnp.float32),
        ],
        mesh=mesh,
        scratch_types=[
            pltpu.VMEM((C,), jnp.float32),
            pltpu.VMEM((L,), jnp.float32),
            pltpu.VMEM((C,), jnp.float32),
            pltpu.VMEM((C,), jnp.float32),
            pltpu.VMEM((W, C), jnp.float32),
            pltpu.VMEM((W, C), jnp.float32),
            pltpu.VMEM((W, C), jnp.float32),
            pltpu.VMEM((W, C), jnp.float32),
        ],
        compiler_params=pltpu.CompilerParams(use_tc_tiling_on_sc=True),
    )
    y1t, y2t = run(x0t, x1t, bn1_weight, bn2_weight, thr)
    y1 = jnp.transpose(y1t, (0, 3, 1, 2))
    y2 = jnp.transpose(y2t, (0, 3, 1, 2))
    return (y1, y2)


# speedup vs baseline: 4.1082x; 1.8274x over previous
import jax
import jax.numpy as jnp
from jax import lax
from jax.experimental import pallas as pl
from jax.experimental.pallas import tpu as pltpu
from jax.experimental.pallas import tpu_sc as plsc

L = 16
HW2 = 32


def _sc_body(x0_hbm, x1_hbm, bn1_hbm, bn2_hbm, thr_hbm,
             y1_hbm, y2_hbm,
             w_v, thr_v, m1_v, m2_v,
             a0, a1, b0, b1, o10, o11, o20, o21,
             sin0, sin1, sout0, sout1):
    nc = 2
    wid = lax.axis_index("s") * nc + lax.axis_index("c")
    B, H, W, C = x0_hbm.shape
    chunks_per_plane = W // HW2
    n_chunks = (B * H * chunks_per_plane) // (nc * L)
    c0 = wid * n_chunks

    abuf = (a0, a1)
    bbuf = (b0, b1)
    o1buf = (o10, o11)
    o2buf = (o20, o21)
    sin = (sin0, sin1)
    sout = (sout0, sout1)

    pltpu.sync_copy(thr_hbm, thr_v)
    thr = thr_v[...]
    pltpu.sync_copy(bn1_hbm, w_v)
    for k in range(C // L):
        m1_v[pl.ds(k * L, L)] = jnp.where(
            jnp.abs(w_v[pl.ds(k * L, L)]) >= thr, 1.0, 0.0)
    pltpu.sync_copy(bn2_hbm, w_v)
    for k in range(C // L):
        m2_v[pl.ds(k * L, L)] = jnp.where(
            jnp.abs(w_v[pl.ds(k * L, L)]) >= thr, 1.0, 0.0)

    def _loc(c):
        p = c // chunks_per_plane
        return p // H, p % H, (c % chunks_per_plane) * HW2

    def _gather_start(c, par):
        b, h, w0 = _loc(c0 + c)
        pltpu.make_async_copy(
            x0_hbm.at[b, h, pl.ds(w0, HW2)], abuf[par], sin[par]).start()
        pltpu.make_async_copy(
            x1_hbm.at[b, h, pl.ds(w0, HW2)], bbuf[par], sin[par]).start()

    def _gather_wait(par):
        pltpu.make_async_copy(
            x0_hbm.at[0, 0, pl.ds(0, HW2)], abuf[par], sin[par]).wait()
        pltpu.make_async_copy(
            x1_hbm.at[0, 0, pl.ds(0, HW2)], bbuf[par], sin[par]).wait()

    def _scatter_start(c, par):
        b, h, w0 = _loc(c0 + c)
        pltpu.make_async_copy(
            o1buf[par], y1_hbm.at[b, h, pl.ds(w0, HW2)], sout[par]).start()
        pltpu.make_async_copy(
            o2buf[par], y2_hbm.at[b, h, pl.ds(w0, HW2)], sout[par]).start()

    def _scatter_wait(par):
        pltpu.make_async_copy(
            o1buf[par], y1_hbm.at[0, 0, pl.ds(0, HW2)], sout[par]).wait()
        pltpu.make_async_copy(
            o2buf[par], y2_hbm.at[0, 0, pl.ds(0, HW2)], sout[par]).wait()

    def _compute(par):
        av_ref, bv_ref = abuf[par], bbuf[par]
        o1_ref, o2_ref = o1buf[par], o2buf[par]

        def kloop(k):
            sl = pl.ds(k * L, L)
            m1 = m1_v[sl] > 0.5
            m2 = m2_v[sl] > 0.5

            def wloop(w):
                av = av_ref[w, sl]
                bv = bv_ref[w, sl]
                o1_ref[w, sl] = jnp.where(m1, av, bv)
                o2_ref[w, sl] = jnp.where(m2, bv, av)

            pl.loop(0, HW2)(wloop)

        pl.loop(0, C // L)(kloop)

    _gather_start(0, 0)

    def step(t):
        for par in range(2):
            c = t * 2 + par

            @pl.when(c + 1 < n_chunks)
            def _():
                _gather_start(c + 1, 1 - par)

            _gather_wait(par)

            @pl.when(c >= 2)
            def _():
                _scatter_wait(par)

            _compute(par)
            _scatter_start(c, par)

    pl.loop(0, n_chunks // 2)(step)
    _scatter_wait(0)
    _scatter_wait(1)


def kernel(x0, x1, bn1_weight, bn2_weight, bn_threshold):
    B, C, H, W = x0.shape
    x0t = jnp.transpose(x0, (0, 2, 3, 1))
    x1t = jnp.transpose(x1, (0, 2, 3, 1))
    thr = jnp.full((L,), bn_threshold, dtype=jnp.float32)

    mesh = plsc.VectorSubcoreMesh(core_axis_name="c", subcore_axis_name="s")
    chunk = pltpu.VMEM((HW2, C), jnp.float32)
    run = pl.kernel(
        _sc_body,
        out_type=[
            jax.ShapeDtypeStruct((B, H, W, C), jnp.float32),
            jax.ShapeDtypeStruct((B, H, W, C), jnp.float32),
        ],
        mesh=mesh,
        scratch_types=[
            pltpu.VMEM((C,), jnp.float32),
            pltpu.VMEM((L,), jnp.float32),
            pltpu.VMEM((C,), jnp.float32),
            pltpu.VMEM((C,), jnp.float32),
            chunk, chunk, chunk, chunk, chunk, chunk, chunk, chunk,
            pltpu.SemaphoreType.DMA,
            pltpu.SemaphoreType.DMA,
            pltpu.SemaphoreType.DMA,
            pltpu.SemaphoreType.DMA,
        ],
        compiler_params=pltpu.CompilerParams(use_tc_tiling_on_sc=True),
    )
    y1t, y2t = run(x0t, x1t, bn1_weight, bn2_weight, thr)
    y1 = jnp.transpose(y1t, (0, 3, 1, 2))
    y2 = jnp.transpose(y2t, (0, 3, 1, 2))
    return (y1, y2)


# speedup vs baseline: 5.5843x; 1.3593x over previous
import jax
import jax.numpy as jnp
from jax import lax
from jax.experimental import pallas as pl
from jax.experimental.pallas import tpu as pltpu
from jax.experimental.pallas import tpu_sc as plsc

L = 16
HW2 = 32
NBUF = 4


def _sc_body(x0_hbm, x1_hbm, bn1_hbm, bn2_hbm, thr_hbm,
             y1_hbm, y2_hbm,
             w_v, thr_v, m1_v, m2_v,
             a0, a1, a2, a3, b0, b1, b2, b3,
             sin0, sin1, sin2, sin3, sout0, sout1, sout2, sout3):
    nc = 2
    wid = lax.axis_index("s") * nc + lax.axis_index("c")
    B, H, W, C = x0_hbm.shape
    cpp = W // HW2
    n_chunks = (B * H * cpp) // (nc * L)
    c0 = wid * n_chunks

    abuf = (a0, a1, a2, a3)
    bbuf = (b0, b1, b2, b3)
    sin = (sin0, sin1, sin2, sin3)
    sout = (sout0, sout1, sout2, sout3)

    pltpu.sync_copy(thr_hbm, thr_v)
    thr = thr_v[...]
    pltpu.sync_copy(bn1_hbm, w_v)
    for k in range(C // L):
        m1_v[pl.ds(k * L, L)] = jnp.where(
            jnp.abs(w_v[pl.ds(k * L, L)]) >= thr, 1.0, 0.0)
    pltpu.sync_copy(bn2_hbm, w_v)
    for k in range(C // L):
        m2_v[pl.ds(k * L, L)] = jnp.where(
            jnp.abs(w_v[pl.ds(k * L, L)]) >= thr, 1.0, 0.0)

    def _loc(c):
        p = c // cpp
        return p // H, p % H, (c % cpp) * HW2

    def _gather_start(c, s):
        b, h, w0 = _loc(c0 + c)
        pltpu.make_async_copy(
            x0_hbm.at[b, h, pl.ds(w0, HW2)], abuf[s], sin[s]).start()
        pltpu.make_async_copy(
            x1_hbm.at[b, h, pl.ds(w0, HW2)], bbuf[s], sin[s]).start()

    def _gather_wait(s):
        pltpu.make_async_copy(
            x0_hbm.at[0, 0, pl.ds(0, HW2)], abuf[s], sin[s]).wait()
        pltpu.make_async_copy(
            x1_hbm.at[0, 0, pl.ds(0, HW2)], bbuf[s], sin[s]).wait()

    def _scatter_start(c, s):
        b, h, w0 = _loc(c0 + c)
        pltpu.make_async_copy(
            abuf[s], y1_hbm.at[b, h, pl.ds(w0, HW2)], sout[s]).start()
        pltpu.make_async_copy(
            bbuf[s], y2_hbm.at[b, h, pl.ds(w0, HW2)], sout[s]).start()

    def _scatter_wait(s):
        pltpu.make_async_copy(
            abuf[s], y1_hbm.at[0, 0, pl.ds(0, HW2)], sout[s]).wait()
        pltpu.make_async_copy(
            bbuf[s], y2_hbm.at[0, 0, pl.ds(0, HW2)], sout[s]).wait()

    def _compute(s):
        av_ref, bv_ref = abuf[s], bbuf[s]

        def kloop(k):
            sl = pl.ds(k * L, L)
            m1 = m1_v[sl] > 0.5
            m2 = m2_v[sl] > 0.5
            for w in range(HW2):
                av = av_ref[w, sl]
                bv = bv_ref[w, sl]
                av_ref[w, sl] = jnp.where(m1, av, bv)
                bv_ref[w, sl] = jnp.where(m2, bv, av)

        pl.loop(0, C // L)(kloop)

    for s in range(NBUF - 1):
        _gather_start(s, s)

    def step(t):
        for s in range(NBUF):
            c = t * NBUF + s
            sprev = (s + NBUF - 1) % NBUF

            _gather_wait(s)
            _compute(s)
            _scatter_start(c, s)

            @pl.when(c + NBUF - 1 < n_chunks)
            def _():
                @pl.when(c >= 1)
                def _():
                    _scatter_wait(sprev)

                _gather_start(c + NBUF - 1, sprev)

    pl.loop(0, n_chunks // NBUF)(step)
    for s in range(NBUF):
        _scatter_wait(s)


def kernel(x0, x1, bn1_weight, bn2_weight, bn_threshold):
    B, C, H, W = x0.shape
    x0t = jnp.transpose(x0, (0, 2, 3, 1))
    x1t = jnp.transpose(x1, (0, 2, 3, 1))
    thr = jnp.full((L,), bn_threshold, dtype=jnp.float32)

    mesh = plsc.VectorSubcoreMesh(core_axis_name="c", subcore_axis_name="s")
    chunk = pltpu.VMEM((HW2, C), jnp.float32)
    run = pl.kernel(
        _sc_body,
        out_type=[
            jax.ShapeDtypeStruct((B, H, W, C), jnp.float32),
            jax.ShapeDtypeStruct((B, H, W, C), jnp.float32),
        ],
        mesh=mesh,
        scratch_types=[
            pltpu.VMEM((C,), jnp.float32),
            pltpu.VMEM((L,), jnp.float32),
            pltpu.VMEM((C,), jnp.float32),
            pltpu.VMEM((C,), jnp.float32),
            chunk, chunk, chunk, chunk, chunk, chunk, chunk, chunk,
            pltpu.SemaphoreType.DMA, pltpu.SemaphoreType.DMA,
            pltpu.SemaphoreType.DMA, pltpu.SemaphoreType.DMA,
            pltpu.SemaphoreType.DMA, pltpu.SemaphoreType.DMA,
            pltpu.SemaphoreType.DMA, pltpu.SemaphoreType.DMA,
        ],
        compiler_params=pltpu.CompilerParams(use_tc_tiling_on_sc=True),
    )
    y1t, y2t = run(x0t, x1t, bn1_weight, bn2_weight, thr)
    return (jnp.transpose(y1t, (0, 3, 1, 2)), jnp.transpose(y2t, (0, 3, 1, 2)))
